# Initial kernel scaffold; baseline (speedup 1.0000x reference)
import jax
import jax.numpy as jnp
from jax.experimental import pallas as pl


def kernel(tokens, h0, c0, h1, c1, emb, W_ih0, W_hh0, b_ih0, b_hh0, W_ih1, W_hh1, b_ih1, b_hh1, W_dec, b_dec, W_o1, b_o1, W_o2, b_o2):
    raise NotImplementedError("write your pallas kernel here")


# speedup vs baseline: 9.0353x; 9.0353x over previous
import functools

import jax
import jax.numpy as jnp
from jax import lax
from jax.experimental import pallas as pl
from jax.experimental.pallas import tpu as pltpu
from jax.experimental.pallas import tpu_sc as plsc

SEQ, BATCH = 32, 16
NTOKEN, NINP, NHID, NHIDLAST = 10000, 128, 256, 128
ODE_STEPS = 4
ROWS = SEQ * BATCH
TILE = 128


def _sc_gather(emb, idx):
    info = plsc.get_sparse_core_info()
    nc, ns = info.num_cores, info.num_subcores
    nw = nc * ns
    b_per_w = ROWS // nw
    mesh = plsc.VectorSubcoreMesh(core_axis_name="c", subcore_axis_name="s")

    @functools.partial(
        pl.kernel,
        mesh=mesh,
        out_type=jax.ShapeDtypeStruct((ROWS, NINP), jnp.float32),
        scratch_types=[
            pltpu.VMEM((b_per_w,), jnp.int32),
            pltpu.VMEM((b_per_w, NINP), jnp.float32),
            pltpu.SemaphoreType.DMA,
        ],
    )
    def gather_kernel(table_hbm, idx_hbm, out_hbm, idx_v, rows_v, sem):
        wid = lax.axis_index("s") * nc + lax.axis_index("c")
        base = wid * b_per_w
        pltpu.sync_copy(idx_hbm.at[pl.ds(base, b_per_w)], idx_v)
        pltpu.async_copy(table_hbm.at[idx_v], rows_v, sem).wait()
        pltpu.sync_copy(rows_v, out_hbm.at[pl.ds(base, b_per_w)])

    return gather_kernel(emb, idx)


def _lstm_body(x_ref, wih0_ref, whh0_ref, b0_ref, wih1_ref, whh1_ref, b1_ref,
               h0_ref, c0_ref, h1_ref, c1_ref,
               y_ref, h0n_ref, c0n_ref, h1n_ref, c1n_ref):
    wih0 = wih0_ref[...]
    whh0 = whh0_ref[...]
    b0 = b0_ref[...]
    wih1 = wih1_ref[...]
    whh1 = whh1_ref[...]
    b1 = b1_ref[...]

    def step(t, carry):
        h0, c0, h1, c1 = carry
        x_t = x_ref[pl.ds(t * BATCH, BATCH), :]
        g = (jnp.dot(x_t, wih0, preferred_element_type=jnp.float32)
             + jnp.dot(h0, whh0, preferred_element_type=jnp.float32) + b0)
        i = jax.nn.sigmoid(g[:, :NHID])
        f = jax.nn.sigmoid(g[:, NHID:2 * NHID])
        gg = jnp.tanh(g[:, 2 * NHID:3 * NHID])
        o = jax.nn.sigmoid(g[:, 3 * NHID:])
        c0 = f * c0 + i * gg
        h0 = o * jnp.tanh(c0)

        g2 = (jnp.dot(h0, wih1, preferred_element_type=jnp.float32)
              + jnp.dot(h1, whh1, preferred_element_type=jnp.float32) + b1)
        i2 = jax.nn.sigmoid(g2[:, :NHIDLAST])
        f2 = jax.nn.sigmoid(g2[:, NHIDLAST:2 * NHIDLAST])
        gg2 = jnp.tanh(g2[:, 2 * NHIDLAST:3 * NHIDLAST])
        o2 = jax.nn.sigmoid(g2[:, 3 * NHIDLAST:])
        c1 = f2 * c1 + i2 * gg2
        h1 = o2 * jnp.tanh(c1)
        y_ref[pl.ds(t * BATCH, BATCH), :] = h1
        return h0, c0, h1, c1

    h0, c0, h1, c1 = lax.fori_loop(
        0, SEQ, step, (h0_ref[...], c0_ref[...], h1_ref[...], c1_ref[...]))
    h0n_ref[...] = h0
    c0n_ref[...] = c0
    h1n_ref[...] = h1
    c1n_ref[...] = c1


def _pre_body(wdecT_ref, wo2T_ref, a_ref, b_ref, g_ref, m_ref, ba_ref):
    a = a_ref[...]
    g_ref[...] = jnp.dot(wdecT_ref[...], a, preferred_element_type=jnp.float32)
    m_ref[...] = jnp.dot(wo2T_ref[...], a, preferred_element_type=jnp.float32)
    ba_ref[...] = jnp.dot(b_ref[...], a, preferred_element_type=jnp.float32)


def _softplus(u):
    return jnp.maximum(u, 0.0) + jnp.log1p(jnp.exp(-jnp.abs(u)))


def _ode_body(y1_ref, wdecT_ref, wo2T_ref, g_ref, m_ref, ba_ref, a0_ref,
              bo1_ref, crow_ref, out_ref):
    y1 = y1_ref[...]
    G = g_ref[...]
    M = m_ref[...]
    bdA = ba_ref[0:1, :]
    v = ba_ref[1:2, :]
    a0 = a0_ref[...]
    b1 = bo1_ref[...]

    x0 = jnp.dot(y1, wdecT_ref[...], preferred_element_type=jnp.float32)

    dt = 1.0 / ODE_STEPS
    p = jnp.dot(y1, G, preferred_element_type=jnp.float32) + bdA
    S = jnp.zeros_like(p)
    for step in range(ODE_STEPS):
        t = step * dt
        u1 = p + t * a0 + b1
        g1 = _softplus(u1)
        k1 = jnp.dot(g1, M, preferred_element_type=jnp.float32) + v
        u2 = p + (dt / 2) * k1 + (t + dt / 2) * a0 + b1
        g2 = _softplus(u2)
        k2 = jnp.dot(g2, M, preferred_element_type=jnp.float32) + v
        u3 = p + (dt / 2) * k2 + (t + dt / 2) * a0 + b1
        g3 = _softplus(u3)
        k3 = jnp.dot(g3, M, preferred_element_type=jnp.float32) + v
        u4 = p + dt * k3 + (t + dt) * a0 + b1
        g4 = _softplus(u4)
        k4 = jnp.dot(g4, M, preferred_element_type=jnp.float32) + v
        S = S + (dt / 6.0) * (g1 + 2.0 * g2 + 2.0 * g3 + g4)
        p = p + (dt / 6.0) * (k1 + 2.0 * k2 + 2.0 * k3 + k4)

    xf = (x0 + jnp.dot(S, wo2T_ref[...], preferred_element_type=jnp.float32)
          + crow_ref[...])
    mx = jnp.max(xf, axis=-1, keepdims=True)
    e = jnp.exp(xf - mx)
    s = jnp.sum(e, axis=-1, keepdims=True)
    out_ref[...] = jnp.log(e / s + 1e-8)


def kernel(tokens, h0, c0, h1, c1, emb, W_ih0, W_hh0, b_ih0, b_hh0,
           W_ih1, W_hh1, b_ih1, b_hh1, W_dec, b_dec, W_o1, b_o1, W_o2, b_o2):
    idx = tokens.reshape(ROWS).astype(jnp.int32)
    x = _sc_gather(emb, idx)

    wih0T = W_ih0.T
    whh0T = W_hh0.T
    b0r = (b_ih0 + b_hh0)[None, :]
    wih1T = W_ih1.T
    whh1T = W_hh1.T
    b1r = (b_ih1 + b_hh1)[None, :]
    wdecT = W_dec.T
    wo2T = W_o2.T
    A = W_o1[:, 1:].T
    a0 = W_o1[:, 0][None, :]
    bo1 = b_o1[None, :]
    crow = (b_dec + b_o2)[None, :]
    bstack = jnp.stack([b_dec, b_o2], axis=0)

    y1, h0n, c0n, h1n, c1n = pl.pallas_call(
        _lstm_body,
        out_shape=[
            jax.ShapeDtypeStruct((ROWS, NHIDLAST), jnp.float32),
            jax.ShapeDtypeStruct((BATCH, NHID), jnp.float32),
            jax.ShapeDtypeStruct((BATCH, NHID), jnp.float32),
            jax.ShapeDtypeStruct((BATCH, NHIDLAST), jnp.float32),
            jax.ShapeDtypeStruct((BATCH, NHIDLAST), jnp.float32),
        ],
    )(x, wih0T, whh0T, b0r, wih1T, whh1T, b1r,
      h0.reshape(BATCH, NHID), c0.reshape(BATCH, NHID),
      h1.reshape(BATCH, NHIDLAST), c1.reshape(BATCH, NHIDLAST))

    G, M, BA = pl.pallas_call(
        _pre_body,
        out_shape=[
            jax.ShapeDtypeStruct((NINP, NINP), jnp.float32),
            jax.ShapeDtypeStruct((NINP, NINP), jnp.float32),
            jax.ShapeDtypeStruct((2, NINP), jnp.float32),
        ],
    )(wdecT, wo2T, A, bstack)

    n_tiles = ROWS // TILE
    out = pl.pallas_call(
        _ode_body,
        grid=(n_tiles,),
        in_specs=[
            pl.BlockSpec((TILE, NINP), lambda i: (i, 0)),
            pl.BlockSpec((NINP, NTOKEN), lambda i: (0, 0)),
            pl.BlockSpec((NINP, NTOKEN), lambda i: (0, 0)),
            pl.BlockSpec((NINP, NINP), lambda i: (0, 0)),
            pl.BlockSpec((NINP, NINP), lambda i: (0, 0)),
            pl.BlockSpec((2, NINP), lambda i: (0, 0)),
            pl.BlockSpec((1, NINP), lambda i: (0, 0)),
            pl.BlockSpec((1, NINP), lambda i: (0, 0)),
            pl.BlockSpec((1, NTOKEN), lambda i: (0, 0)),
        ],
        out_specs=pl.BlockSpec((TILE, NTOKEN), lambda i: (i, 0)),
        out_shape=jax.ShapeDtypeStruct((ROWS, NTOKEN), jnp.float32),
    )(y1, wdecT, wo2T, G, M, BA, a0, bo1, crow)

    return (out.reshape(SEQ, BATCH, NTOKEN),
            h0n[None], c0n[None], h1n[None], c1n[None])


# speedup vs baseline: 9.8116x; 1.0859x over previous
import functools

import jax
import jax.numpy as jnp
from jax import lax
from jax.experimental import pallas as pl
from jax.experimental.pallas import tpu as pltpu
from jax.experimental.pallas import tpu_sc as plsc

SEQ, BATCH = 32, 16
NTOKEN, NINP, NHID, NHIDLAST = 10000, 128, 256, 128
ODE_STEPS = 4
ROWS = SEQ * BATCH
TILE = 128


def _sc_gather(emb, idx):
    info = plsc.get_sparse_core_info()
    nc, ns = info.num_cores, info.num_subcores
    nw = nc * ns
    b_per_w = ROWS // nw
    mesh = plsc.VectorSubcoreMesh(core_axis_name="c", subcore_axis_name="s")

    @functools.partial(
        pl.kernel,
        mesh=mesh,
        out_type=jax.ShapeDtypeStruct((ROWS, NINP), jnp.float32),
        scratch_types=[
            pltpu.VMEM((b_per_w,), jnp.int32),
            pltpu.VMEM((b_per_w, NINP), jnp.float32),
            pltpu.SemaphoreType.DMA,
        ],
    )
    def gather_kernel(table_hbm, idx_hbm, out_hbm, idx_v, rows_v, sem):
        wid = lax.axis_index("s") * nc + lax.axis_index("c")
        base = wid * b_per_w
        pltpu.sync_copy(idx_hbm.at[pl.ds(base, b_per_w)], idx_v)
        pltpu.async_copy(table_hbm.at[idx_v], rows_v, sem).wait()
        pltpu.sync_copy(rows_v, out_hbm.at[pl.ds(base, b_per_w)])

    return gather_kernel(emb, idx)


def _softplus(u):
    return jnp.maximum(u, 0.0) + jnp.log1p(jnp.exp(-jnp.abs(u)))


def _fused_body(x_ref, wih0_ref, whh0_ref, b0_ref, wih1_ref, whh1_ref, b1_ref,
                h0_ref, c0_ref, h1_ref, c1_ref,
                wdecT_ref, wo2T_ref, a_ref, bstack_ref, a0_ref, bo1_ref,
                crow_ref,
                out_ref, h0n_ref, c0n_ref, h1n_ref, c1n_ref,
                y1_s, g_s, m_s, ba_s, xw_s):
    pid = pl.program_id(0)

    @pl.when(pid == 0)
    def _prologue():
        a = a_ref[...]
        g_s[...] = jnp.dot(wdecT_ref[...], a, preferred_element_type=jnp.float32)
        m_s[...] = jnp.dot(wo2T_ref[...], a, preferred_element_type=jnp.float32)
        ba_s[...] = jnp.dot(bstack_ref[...], a, preferred_element_type=jnp.float32)

        whh0 = whh0_ref[...]
        b0 = b0_ref[...]
        wih1 = wih1_ref[...]
        whh1 = whh1_ref[...]
        b1 = b1_ref[...]
        xw_s[...] = jnp.dot(x_ref[...], wih0_ref[...],
                            preferred_element_type=jnp.float32) + b0

        def step(t, carry):
            h0, c0, h1, c1 = carry
            g = (xw_s[pl.ds(t * BATCH, BATCH), :]
                 + jnp.dot(h0, whh0, preferred_element_type=jnp.float32))
            i = jax.nn.sigmoid(g[:, :NHID])
            f = jax.nn.sigmoid(g[:, NHID:2 * NHID])
            gg = jnp.tanh(g[:, 2 * NHID:3 * NHID])
            o = jax.nn.sigmoid(g[:, 3 * NHID:])
            c0 = f * c0 + i * gg
            h0 = o * jnp.tanh(c0)

            g2 = (jnp.dot(h0, wih1, preferred_element_type=jnp.float32)
                  + jnp.dot(h1, whh1, preferred_element_type=jnp.float32) + b1)
            i2 = jax.nn.sigmoid(g2[:, :NHIDLAST])
            f2 = jax.nn.sigmoid(g2[:, NHIDLAST:2 * NHIDLAST])
            gg2 = jnp.tanh(g2[:, 2 * NHIDLAST:3 * NHIDLAST])
            o2 = jax.nn.sigmoid(g2[:, 3 * NHIDLAST:])
            c1 = f2 * c1 + i2 * gg2
            h1 = o2 * jnp.tanh(c1)
            y1_s[pl.ds(t * BATCH, BATCH), :] = h1
            return h0, c0, h1, c1

        h0, c0, h1, c1 = lax.fori_loop(
            0, SEQ, step,
            (h0_ref[...], c0_ref[...], h1_ref[...], c1_ref[...]))
        h0n_ref[...] = h0
        c0n_ref[...] = c0
        h1n_ref[...] = h1
        c1n_ref[...] = c1

    y1 = y1_s[pl.ds(pid * TILE, TILE), :]
    G = g_s[...]
    M = m_s[...]
    bdA = ba_s[0:1, :]
    v = ba_s[1:2, :]
    a0 = a0_ref[...]
    b1o = bo1_ref[...]

    x0 = jnp.dot(y1, wdecT_ref[...], preferred_element_type=jnp.float32)

    dt = 1.0 / ODE_STEPS
    p = jnp.dot(y1, G, preferred_element_type=jnp.float32) + bdA
    S = jnp.zeros_like(p)
    for step_i in range(ODE_STEPS):
        t = step_i * dt
        u1 = p + t * a0 + b1o
        g1 = _softplus(u1)
        k1 = jnp.dot(g1, M, preferred_element_type=jnp.float32) + v
        u2 = p + (dt / 2) * k1 + (t + dt / 2) * a0 + b1o
        g2 = _softplus(u2)
        k2 = jnp.dot(g2, M, preferred_element_type=jnp.float32) + v
        u3 = p + (dt / 2) * k2 + (t + dt / 2) * a0 + b1o
        g3 = _softplus(u3)
        k3 = jnp.dot(g3, M, preferred_element_type=jnp.float32) + v
        u4 = p + dt * k3 + (t + dt) * a0 + b1o
        g4 = _softplus(u4)
        k4 = jnp.dot(g4, M, preferred_element_type=jnp.float32) + v
        S = S + (dt / 6.0) * (g1 + 2.0 * g2 + 2.0 * g3 + g4)
        p = p + (dt / 6.0) * (k1 + 2.0 * k2 + 2.0 * k3 + k4)

    xf = (x0 + jnp.dot(S, wo2T_ref[...], preferred_element_type=jnp.float32)
          + crow_ref[...])
    mx = jnp.max(xf, axis=-1, keepdims=True)
    e = jnp.exp(xf - mx)
    s = jnp.sum(e, axis=-1, keepdims=True)
    out_ref[...] = jnp.log(e / s + 1e-8)


def kernel(tokens, h0, c0, h1, c1, emb, W_ih0, W_hh0, b_ih0, b_hh0,
           W_ih1, W_hh1, b_ih1, b_hh1, W_dec, b_dec, W_o1, b_o1, W_o2, b_o2):
    idx = tokens.reshape(ROWS).astype(jnp.int32)
    x = _sc_gather(emb, idx)

    wih0T = W_ih0.T
    whh0T = W_hh0.T
    b0r = (b_ih0 + b_hh0)[None, :]
    wih1T = W_ih1.T
    whh1T = W_hh1.T
    b1r = (b_ih1 + b_hh1)[None, :]
    wdecT = W_dec.T
    wo2T = W_o2.T
    A = W_o1[:, 1:].T
    a0 = W_o1[:, 0][None, :]
    bo1 = b_o1[None, :]
    crow = (b_dec + b_o2)[None, :]
    bstack = jnp.stack([b_dec, b_o2], axis=0)

    n_tiles = ROWS // TILE
    const = lambda i: (0, 0)
    out, h0n, c0n, h1n, c1n = pl.pallas_call(
        _fused_body,
        grid=(n_tiles,),
        in_specs=[
            pl.BlockSpec((ROWS, NINP), const),
            pl.BlockSpec((NINP, 4 * NHID), const),
            pl.BlockSpec((NHID, 4 * NHID), const),
            pl.BlockSpec((1, 4 * NHID), const),
            pl.BlockSpec((NHID, 4 * NHIDLAST), const),
            pl.BlockSpec((NHIDLAST, 4 * NHIDLAST), const),
            pl.BlockSpec((1, 4 * NHIDLAST), const),
            pl.BlockSpec((BATCH, NHID), const),
            pl.BlockSpec((BATCH, NHID), const),
            pl.BlockSpec((BATCH, NHIDLAST), const),
            pl.BlockSpec((BATCH, NHIDLAST), const),
            pl.BlockSpec((NINP, NTOKEN), const),
            pl.BlockSpec((NINP, NTOKEN), const),
            pl.BlockSpec((NTOKEN, NINP), const),
            pl.BlockSpec((2, NTOKEN), const),
            pl.BlockSpec((1, NINP), const),
            pl.BlockSpec((1, NINP), const),
            pl.BlockSpec((1, NTOKEN), const),
        ],
        out_specs=[
            pl.BlockSpec((TILE, NTOKEN), lambda i: (i, 0)),
            pl.BlockSpec((BATCH, NHID), const),
            pl.BlockSpec((BATCH, NHID), const),
            pl.BlockSpec((BATCH, NHIDLAST), const),
            pl.BlockSpec((BATCH, NHIDLAST), const),
        ],
        out_shape=[
            jax.ShapeDtypeStruct((ROWS, NTOKEN), jnp.float32),
            jax.ShapeDtypeStruct((BATCH, NHID), jnp.float32),
            jax.ShapeDtypeStruct((BATCH, NHID), jnp.float32),
            jax.ShapeDtypeStruct((BATCH, NHIDLAST), jnp.float32),
            jax.ShapeDtypeStruct((BATCH, NHIDLAST), jnp.float32),
        ],
        scratch_shapes=[
            pltpu.VMEM((ROWS, NHIDLAST), jnp.float32),
            pltpu.VMEM((NINP, NINP), jnp.float32),
            pltpu.VMEM((NINP, NINP), jnp.float32),
            pltpu.VMEM((2, NINP), jnp.float32),
            pltpu.VMEM((ROWS, 4 * NHID), jnp.float32),
        ],
    )(x, wih0T, whh0T, b0r, wih1T, whh1T, b1r,
      h0.reshape(BATCH, NHID), c0.reshape(BATCH, NHID),
      h1.reshape(BATCH, NHIDLAST), c1.reshape(BATCH, NHIDLAST),
      wdecT, wo2T, A, bstack, a0, bo1, crow)

    return (out.reshape(SEQ, BATCH, NTOKEN),
            h0n[None], c0n[None], h1n[None], c1n[None])


# speedup vs baseline: 10.5001x; 1.0702x over previous
import functools

import jax
import jax.numpy as jnp
from jax import lax
from jax.experimental import pallas as pl
from jax.experimental.pallas import tpu as pltpu
from jax.experimental.pallas import tpu_sc as plsc

SEQ, BATCH = 32, 16
NTOKEN, NINP, NHID, NHIDLAST = 10000, 128, 256, 128
ODE_STEPS = 4
ROWS = SEQ * BATCH
TILE = 128

_NT = (((1,), (1,)), ((), ()))


def _sc_gather(emb, idx):
    info = plsc.get_sparse_core_info()
    nc, ns = info.num_cores, info.num_subcores
    nw = nc * ns
    b_per_w = ROWS // nw
    mesh = plsc.VectorSubcoreMesh(core_axis_name="c", subcore_axis_name="s")

    @functools.partial(
        pl.kernel,
        mesh=mesh,
        out_type=jax.ShapeDtypeStruct((ROWS, NINP), jnp.float32),
        scratch_types=[
            pltpu.VMEM((b_per_w,), jnp.int32),
            pltpu.VMEM((b_per_w, NINP), jnp.float32),
            pltpu.SemaphoreType.DMA,
        ],
    )
    def gather_kernel(table_hbm, idx_hbm, out_hbm, idx_v, rows_v, sem):
        wid = lax.axis_index("s") * nc + lax.axis_index("c")
        base = wid * b_per_w
        pltpu.sync_copy(idx_hbm.at[pl.ds(base, b_per_w)], idx_v)
        pltpu.async_copy(table_hbm.at[idx_v], rows_v, sem).wait()
        pltpu.sync_copy(rows_v, out_hbm.at[pl.ds(base, b_per_w)])

    return gather_kernel(emb, idx)


def _softplus(u):
    return jnp.maximum(u, 0.0) + jnp.log1p(jnp.exp(-jnp.abs(u)))


def _fused_body(x_ref, wih0_ref, whh0_ref, b0_ref, wih1_ref, whh1_ref, b1_ref,
                h0_ref, c0_ref, h1_ref, c1_ref,
                wdec_ref, wo2_ref, wo1_ref, bstackT_ref, a0_ref, bo1_ref,
                crow_ref,
                out_ref, h0n_ref, c0n_ref, h1n_ref, c1n_ref,
                y1_s, g_s, m_s, ba_s, xw_s):
    pid = pl.program_id(0)

    @pl.when(pid == 0)
    def _prologue():
        wo1s = wo1_ref[:, 1:NTOKEN + 1]
        g_s[...] = jnp.dot(wo1s, wdec_ref[...],
                           preferred_element_type=jnp.float32)
        m_s[...] = jnp.dot(wo1s, wo2_ref[...],
                           preferred_element_type=jnp.float32)
        bac = jnp.dot(wo1s, bstackT_ref[...],
                      preferred_element_type=jnp.float32)
        ba_s[...] = jnp.swapaxes(bac, 0, 1)

        whh0 = whh0_ref[...]
        b0 = b0_ref[...]
        wih1 = wih1_ref[...]
        whh1 = whh1_ref[...]
        b1 = b1_ref[...]
        xw_s[...] = jnp.dot(x_ref[...], wih0_ref[...],
                            preferred_element_type=jnp.float32) + b0

        def step(t, carry):
            h0, c0, h1, c1 = carry
            g = (xw_s[pl.ds(t * BATCH, BATCH), :]
                 + jnp.dot(h0, whh0, preferred_element_type=jnp.float32))
            i = jax.nn.sigmoid(g[:, :NHID])
            f = jax.nn.sigmoid(g[:, NHID:2 * NHID])
            gg = jnp.tanh(g[:, 2 * NHID:3 * NHID])
            o = jax.nn.sigmoid(g[:, 3 * NHID:])
            c0 = f * c0 + i * gg
            h0 = o * jnp.tanh(c0)

            g2 = (jnp.dot(h0, wih1, preferred_element_type=jnp.float32)
                  + jnp.dot(h1, whh1, preferred_element_type=jnp.float32) + b1)
            i2 = jax.nn.sigmoid(g2[:, :NHIDLAST])
            f2 = jax.nn.sigmoid(g2[:, NHIDLAST:2 * NHIDLAST])
            gg2 = jnp.tanh(g2[:, 2 * NHIDLAST:3 * NHIDLAST])
            o2 = jax.nn.sigmoid(g2[:, 3 * NHIDLAST:])
            c1 = f2 * c1 + i2 * gg2
            h1 = o2 * jnp.tanh(c1)
            y1_s[pl.ds(t * BATCH, BATCH), :] = h1
            return h0, c0, h1, c1

        h0, c0, h1, c1 = lax.fori_loop(
            0, SEQ, step,
            (h0_ref[...], c0_ref[...], h1_ref[...], c1_ref[...]))
        h0n_ref[...] = h0
        c0n_ref[...] = c0
        h1n_ref[...] = h1
        c1n_ref[...] = c1

    y1 = y1_s[pl.ds(pid * TILE, TILE), :]
    Gt = g_s[...]
    Mt = m_s[...]
    bdA = ba_s[0:1, :]
    v = ba_s[1:2, :]
    a0 = a0_ref[...]
    b1o = bo1_ref[...]

    x0 = lax.dot_general(y1, wdec_ref[...], _NT,
                         preferred_element_type=jnp.float32)

    dt = 1.0 / ODE_STEPS
    p = lax.dot_general(y1, Gt, _NT, preferred_element_type=jnp.float32) + bdA
    S = jnp.zeros_like(p)
    for step_i in range(ODE_STEPS):
        t = step_i * dt
        u1 = p + t * a0 + b1o
        g1 = _softplus(u1)
        k1 = lax.dot_general(g1, Mt, _NT,
                             preferred_element_type=jnp.float32) + v
        u2 = p + (dt / 2) * k1 + (t + dt / 2) * a0 + b1o
        g2 = _softplus(u2)
        k2 = lax.dot_general(g2, Mt, _NT,
                             preferred_element_type=jnp.float32) + v
        u3 = p + (dt / 2) * k2 + (t + dt / 2) * a0 + b1o
        g3 = _softplus(u3)
        k3 = lax.dot_general(g3, Mt, _NT,
                             preferred_element_type=jnp.float32) + v
        u4 = p + dt * k3 + (t + dt) * a0 + b1o
        g4 = _softplus(u4)
        k4 = lax.dot_general(g4, Mt, _NT,
                             preferred_element_type=jnp.float32) + v
        S = S + (dt / 6.0) * (g1 + 2.0 * g2 + 2.0 * g3 + g4)
        p = p + (dt / 6.0) * (k1 + 2.0 * k2 + 2.0 * k3 + k4)

    xf = (x0 + lax.dot_general(S, wo2_ref[...], _NT,
                               preferred_element_type=jnp.float32)
          + crow_ref[...])
    mx = jnp.max(xf, axis=-1, keepdims=True)
    e = jnp.exp(xf - mx)
    s = jnp.sum(e, axis=-1, keepdims=True)
    out_ref[...] = jnp.log(e / s + 1e-8)


def kernel(tokens, h0, c0, h1, c1, emb, W_ih0, W_hh0, b_ih0, b_hh0,
           W_ih1, W_hh1, b_ih1, b_hh1, W_dec, b_dec, W_o1, b_o1, W_o2, b_o2):
    idx = tokens.reshape(ROWS).astype(jnp.int32)
    x = _sc_gather(emb, idx)

    wih0T = W_ih0.T
    whh0T = W_hh0.T
    b0r = (b_ih0 + b_hh0)[None, :]
    wih1T = W_ih1.T
    whh1T = W_hh1.T
    b1r = (b_ih1 + b_hh1)[None, :]
    a0 = W_o1[:, 0][None, :]
    bo1 = b_o1[None, :]
    crow = (b_dec + b_o2)[None, :]
    bstackT = jnp.stack([b_dec, b_o2], axis=1)

    n_tiles = ROWS // TILE
    const = lambda i: (0, 0)
    out, h0n, c0n, h1n, c1n = pl.pallas_call(
        _fused_body,
        grid=(n_tiles,),
        in_specs=[
            pl.BlockSpec((ROWS, NINP), const),
            pl.BlockSpec((NINP, 4 * NHID), const),
            pl.BlockSpec((NHID, 4 * NHID), const),
            pl.BlockSpec((1, 4 * NHID), const),
            pl.BlockSpec((NHID, 4 * NHIDLAST), const),
            pl.BlockSpec((NHIDLAST, 4 * NHIDLAST), const),
            pl.BlockSpec((1, 4 * NHIDLAST), const),
            pl.BlockSpec((BATCH, NHID), const),
            pl.BlockSpec((BATCH, NHID), const),
            pl.BlockSpec((BATCH, NHIDLAST), const),
            pl.BlockSpec((BATCH, NHIDLAST), const),
            pl.BlockSpec((NTOKEN, NINP), const),
            pl.BlockSpec((NTOKEN, NINP), const),
            pl.BlockSpec((NINP, NTOKEN + 1), const),
            pl.BlockSpec((NTOKEN, 2), const),
            pl.BlockSpec((1, NINP), const),
            pl.BlockSpec((1, NINP), const),
            pl.BlockSpec((1, NTOKEN), const),
        ],
        out_specs=[
            pl.BlockSpec((TILE, NTOKEN), lambda i: (i, 0)),
            pl.BlockSpec((BATCH, NHID), const),
            pl.BlockSpec((BATCH, NHID), const),
            pl.BlockSpec((BATCH, NHIDLAST), const),
            pl.BlockSpec((BATCH, NHIDLAST), const),
        ],
        out_shape=[
            jax.ShapeDtypeStruct((ROWS, NTOKEN), jnp.float32),
            jax.ShapeDtypeStruct((BATCH, NHID), jnp.float32),
            jax.ShapeDtypeStruct((BATCH, NHID), jnp.float32),
            jax.ShapeDtypeStruct((BATCH, NHIDLAST), jnp.float32),
            jax.ShapeDtypeStruct((BATCH, NHIDLAST), jnp.float32),
        ],
        scratch_shapes=[
            pltpu.VMEM((ROWS, NHIDLAST), jnp.float32),
            pltpu.VMEM((NINP, NINP), jnp.float32),
            pltpu.VMEM((NINP, NINP), jnp.float32),
            pltpu.VMEM((2, NINP), jnp.float32),
            pltpu.VMEM((ROWS, 4 * NHID), jnp.float32),
        ],
    )(x, wih0T, whh0T, b0r, wih1T, whh1T, b1r,
      h0.reshape(BATCH, NHID), c0.reshape(BATCH, NHID),
      h1.reshape(BATCH, NHIDLAST), c1.reshape(BATCH, NHIDLAST),
      W_dec, W_o2, W_o1, bstackT, a0, bo1, crow)

    return (out.reshape(SEQ, BATCH, NTOKEN),
            h0n[None], c0n[None], h1n[None], c1n[None])


# speedup vs baseline: 10.7446x; 1.0233x over previous
import functools

import jax
import jax.numpy as jnp
from jax import lax
from jax.experimental import pallas as pl
from jax.experimental.pallas import tpu as pltpu
from jax.experimental.pallas import tpu_sc as plsc

SEQ, BATCH = 32, 16
NTOKEN, NINP, NHID, NHIDLAST = 10000, 128, 256, 128
ODE_STEPS = 4
ROWS = SEQ * BATCH
TILE = 128

_NT = (((1,), (1,)), ((), ()))


def _sc_gather(emb, idx):
    info = plsc.get_sparse_core_info()
    nc, ns = info.num_cores, info.num_subcores
    nw = nc * ns
    b_per_w = ROWS // nw
    mesh = plsc.VectorSubcoreMesh(core_axis_name="c", subcore_axis_name="s")

    @functools.partial(
        pl.kernel,
        mesh=mesh,
        out_type=jax.ShapeDtypeStruct((ROWS, NINP), jnp.float32),
        scratch_types=[
            pltpu.VMEM((b_per_w,), jnp.int32),
            pltpu.VMEM((b_per_w, NINP), jnp.float32),
            pltpu.SemaphoreType.DMA,
        ],
    )
    def gather_kernel(table_hbm, idx_hbm, out_hbm, idx_v, rows_v, sem):
        wid = lax.axis_index("s") * nc + lax.axis_index("c")
        base = wid * b_per_w
        pltpu.sync_copy(idx_hbm.at[pl.ds(base, b_per_w)], idx_v)
        pltpu.async_copy(table_hbm.at[idx_v], rows_v, sem).wait()
        pltpu.sync_copy(rows_v, out_hbm.at[pl.ds(base, b_per_w)])

    return gather_kernel(emb, idx)


def _softplus(u):
    return jnp.maximum(u, 0.0) + jnp.log1p(jnp.exp(-jnp.abs(u)))


def _fused_body(x_ref, wih0_ref, whh0_ref, b0_ref, wih1_ref, whh1_ref, b1_ref,
                h0_ref, c0_ref, h1_ref, c1_ref,
                wdec_ref, wo2_ref, wo1_ref, bstackT_ref, a0_ref, bo1_ref,
                crow_ref,
                out_ref, h0n_ref, c0n_ref, h1n_ref, c1n_ref,
                y1_s, g_s, m_s, ba_s, xw_s):
    pid = pl.program_id(0)

    @pl.when(pid == 0)
    def _prologue():
        wo1s = wo1_ref[:, 1:NTOKEN + 1]
        g_s[...] = jnp.dot(wo1s, wdec_ref[...],
                           preferred_element_type=jnp.float32)
        m_s[...] = jnp.dot(wo1s, wo2_ref[...],
                           preferred_element_type=jnp.float32)
        bac = jnp.dot(wo1s, bstackT_ref[...],
                      preferred_element_type=jnp.float32)
        ba_s[...] = jnp.swapaxes(bac, 0, 1)

        whh0 = whh0_ref[...]
        b0 = b0_ref[...]
        wih1 = wih1_ref[...]
        whh1 = whh1_ref[...]
        b1 = b1_ref[...]
        xw_s[...] = jnp.dot(x_ref[...], wih0_ref[...],
                            preferred_element_type=jnp.float32) + b0

        def l0_step(xw_t, h0, c0):
            g = xw_t + jnp.dot(h0, whh0, preferred_element_type=jnp.float32)
            i = jax.nn.sigmoid(g[:, :NHID])
            f = jax.nn.sigmoid(g[:, NHID:2 * NHID])
            gg = jnp.tanh(g[:, 2 * NHID:3 * NHID])
            o = jax.nn.sigmoid(g[:, 3 * NHID:])
            c0 = f * c0 + i * gg
            return o * jnp.tanh(c0), c0

        def l1_step(y0, h1, c1):
            g2 = (jnp.dot(y0, wih1, preferred_element_type=jnp.float32)
                  + jnp.dot(h1, whh1, preferred_element_type=jnp.float32) + b1)
            i2 = jax.nn.sigmoid(g2[:, :NHIDLAST])
            f2 = jax.nn.sigmoid(g2[:, NHIDLAST:2 * NHIDLAST])
            gg2 = jnp.tanh(g2[:, 2 * NHIDLAST:3 * NHIDLAST])
            o2 = jax.nn.sigmoid(g2[:, 3 * NHIDLAST:])
            c1 = f2 * c1 + i2 * gg2
            return o2 * jnp.tanh(c1), c1

        h0, c0 = l0_step(xw_s[0:BATCH, :], h0_ref[...], c0_ref[...])

        def step(t, carry):
            h0, c0, h1, c1 = carry
            nh1, nc1 = l1_step(h0, h1, c1)
            nh0, nc0 = l0_step(xw_s[pl.ds(t * BATCH, BATCH), :], h0, c0)
            y1_s[pl.ds((t - 1) * BATCH, BATCH), :] = nh1
            return nh0, nc0, nh1, nc1

        h0, c0, h1, c1 = lax.fori_loop(
            1, SEQ, step, (h0, c0, h1_ref[...], c1_ref[...]))
        h1, c1 = l1_step(h0, h1, c1)
        y1_s[pl.ds((SEQ - 1) * BATCH, BATCH), :] = h1
        h0n_ref[...] = h0
        c0n_ref[...] = c0
        h1n_ref[...] = h1
        c1n_ref[...] = c1

    y1 = y1_s[pl.ds(pid * TILE, TILE), :]
    Gt = g_s[...]
    Mt = m_s[...]
    bdA = ba_s[0:1, :]
    v = ba_s[1:2, :]
    a0 = a0_ref[...]
    b1o = bo1_ref[...]

    x0 = lax.dot_general(y1, wdec_ref[...], _NT,
                         preferred_element_type=jnp.float32)

    dt = 1.0 / ODE_STEPS
    p = lax.dot_general(y1, Gt, _NT, preferred_element_type=jnp.float32) + bdA
    S = jnp.zeros_like(p)
    for step_i in range(ODE_STEPS):
        t = step_i * dt
        u1 = p + t * a0 + b1o
        g1 = _softplus(u1)
        k1 = lax.dot_general(g1, Mt, _NT,
                             preferred_element_type=jnp.float32) + v
        u2 = p + (dt / 2) * k1 + (t + dt / 2) * a0 + b1o
        g2 = _softplus(u2)
        k2 = lax.dot_general(g2, Mt, _NT,
                             preferred_element_type=jnp.float32) + v
        u3 = p + (dt / 2) * k2 + (t + dt / 2) * a0 + b1o
        g3 = _softplus(u3)
        k3 = lax.dot_general(g3, Mt, _NT,
                             preferred_element_type=jnp.float32) + v
        u4 = p + dt * k3 + (t + dt) * a0 + b1o
        g4 = _softplus(u4)
        k4 = lax.dot_general(g4, Mt, _NT,
                             preferred_element_type=jnp.float32) + v
        S = S + (dt / 6.0) * (g1 + 2.0 * g2 + 2.0 * g3 + g4)
        p = p + (dt / 6.0) * (k1 + 2.0 * k2 + 2.0 * k3 + k4)

    xf = (x0 + lax.dot_general(S, wo2_ref[...], _NT,
                               preferred_element_type=jnp.float32)
          + crow_ref[...])
    mx = jnp.max(xf, axis=-1, keepdims=True)
    e = jnp.exp(xf - mx)
    s = jnp.sum(e, axis=-1, keepdims=True)
    out_ref[...] = jnp.log(e + 1e-8 * s) - jnp.log(s)


def kernel(tokens, h0, c0, h1, c1, emb, W_ih0, W_hh0, b_ih0, b_hh0,
           W_ih1, W_hh1, b_ih1, b_hh1, W_dec, b_dec, W_o1, b_o1, W_o2, b_o2):
    idx = tokens.reshape(ROWS).astype(jnp.int32)
    x = _sc_gather(emb, idx)

    wih0T = W_ih0.T
    whh0T = W_hh0.T
    b0r = (b_ih0 + b_hh0)[None, :]
    wih1T = W_ih1.T
    whh1T = W_hh1.T
    b1r = (b_ih1 + b_hh1)[None, :]
    a0 = W_o1[:, 0][None, :]
    bo1 = b_o1[None, :]
    crow = (b_dec + b_o2)[None, :]
    bstackT = jnp.stack([b_dec, b_o2], axis=1)

    n_tiles = ROWS // TILE
    const = lambda i: (0, 0)
    out, h0n, c0n, h1n, c1n = pl.pallas_call(
        _fused_body,
        grid=(n_tiles,),
        in_specs=[
            pl.BlockSpec((ROWS, NINP), const),
            pl.BlockSpec((NINP, 4 * NHID), const),
            pl.BlockSpec((NHID, 4 * NHID), const),
            pl.BlockSpec((1, 4 * NHID), const),
            pl.BlockSpec((NHID, 4 * NHIDLAST), const),
            pl.BlockSpec((NHIDLAST, 4 * NHIDLAST), const),
            pl.BlockSpec((1, 4 * NHIDLAST), const),
            pl.BlockSpec((BATCH, NHID), const),
            pl.BlockSpec((BATCH, NHID), const),
            pl.BlockSpec((BATCH, NHIDLAST), const),
            pl.BlockSpec((BATCH, NHIDLAST), const),
            pl.BlockSpec((NTOKEN, NINP), const),
            pl.BlockSpec((NTOKEN, NINP), const),
            pl.BlockSpec((NINP, NTOKEN + 1), const),
            pl.BlockSpec((NTOKEN, 2), const),
            pl.BlockSpec((1, NINP), const),
            pl.BlockSpec((1, NINP), const),
            pl.BlockSpec((1, NTOKEN), const),
        ],
        out_specs=[
            pl.BlockSpec((TILE, NTOKEN), lambda i: (i, 0)),
            pl.BlockSpec((BATCH, NHID), const),
            pl.BlockSpec((BATCH, NHID), const),
            pl.BlockSpec((BATCH, NHIDLAST), const),
            pl.BlockSpec((BATCH, NHIDLAST), const),
        ],
        out_shape=[
            jax.ShapeDtypeStruct((ROWS, NTOKEN), jnp.float32),
            jax.ShapeDtypeStruct((BATCH, NHID), jnp.float32),
            jax.ShapeDtypeStruct((BATCH, NHID), jnp.float32),
            jax.ShapeDtypeStruct((BATCH, NHIDLAST), jnp.float32),
            jax.ShapeDtypeStruct((BATCH, NHIDLAST), jnp.float32),
        ],
        scratch_shapes=[
            pltpu.VMEM((ROWS, NHIDLAST), jnp.float32),
            pltpu.VMEM((NINP, NINP), jnp.float32),
            pltpu.VMEM((NINP, NINP), jnp.float32),
            pltpu.VMEM((2, NINP), jnp.float32),
            pltpu.VMEM((ROWS, 4 * NHID), jnp.float32),
        ],
    )(x, wih0T, whh0T, b0r, wih1T, whh1T, b1r,
      h0.reshape(BATCH, NHID), c0.reshape(BATCH, NHID),
      h1.reshape(BATCH, NHIDLAST), c1.reshape(BATCH, NHIDLAST),
      W_dec, W_o2, W_o1, bstackT, a0, bo1, crow)

    return (out.reshape(SEQ, BATCH, NTOKEN),
            h0n[None], c0n[None], h1n[None], c1n[None])
